# Initial kernel scaffold; baseline (speedup 1.0000x reference)
import jax
import jax.numpy as jnp
from jax.experimental import pallas as pl


def kernel(x, edge_index, edge_w0, edge_b0, edge_w1, edge_b1, bias0, bias1, cls_w1, cls_b1, cls_w2, cls_b2):
    raise NotImplementedError("write your pallas kernel here")


# speedup vs baseline: 13.2603x; 13.2603x over previous
import jax
import jax.numpy as jnp
from jax import lax
from jax.experimental import pallas as pl
from jax.experimental.pallas import tpu as pltpu
from jax.experimental.pallas import tpu_sc as plsc

_NC = 2
_NS = 16
_NW = _NC * _NS
_CH = 128
_NBUF = 8

_HI = lax.Precision.HIGHEST


def _leaky(v):
    return jnp.where(v >= 0, v, 0.01 * v)


def _segment_accumulate(table, src3, dst3, rows_tot, d, j_chunks):
    acc_rows = (rows_tot // (_NS * _CH) + 1) * (_NS * _CH)
    zrows = acc_rows // _NS
    ngroups = j_chunks // _NBUF
    mesh = plsc.VectorSubcoreMesh(core_axis_name="c", subcore_axis_name="s",
                                  num_cores=_NC, num_subcores=_NS)

    def body(table_hbm, src_hbm, dst_hbm, out_hbm,
             sidx_v, didx_v, rows_v, zbuf_v, gsem, ssem, acc_sh):
        cid = lax.axis_index("c")
        sid = lax.axis_index("s")
        wid = sid * _NC + cid

        zv = jnp.zeros((16,), jnp.float32)
        for i in range(_CH):
            zbuf_v[i] = zv
        for t in range(zrows // _CH):
            pltpu.sync_copy(zbuf_v, acc_sh.at[pl.ds(sid * zrows + t * _CH, _CH)])

        pltpu.sync_copy(src_hbm.at[wid], sidx_v)
        pltpu.sync_copy(dst_hbm.at[wid], didx_v)
        plsc.subcore_barrier()

        def group(g, carry):
            base = g * _NBUF
            gd = [pltpu.async_copy(table_hbm.at[sidx_v.at[base + b]],
                                   rows_v.at[b], gsem)
                  for b in range(_NBUF)]
            for dsc in gd:
                dsc.wait()
            sd = [pltpu.async_copy(rows_v.at[b], acc_sh.at[didx_v.at[base + b]],
                                   ssem, add=True)
                  for b in range(_NBUF)]
            for dsc in sd:
                dsc.wait()
            return carry
        lax.fori_loop(0, ngroups, group, 0)
        plsc.subcore_barrier()

        pltpu.sync_copy(acc_sh.at[pl.ds(sid * zrows, zrows)],
                        out_hbm.at[pl.ds(cid * acc_rows + sid * zrows, zrows)])

    fn = pl.kernel(
        body,
        out_type=jax.ShapeDtypeStruct((_NC * acc_rows, d), jnp.float32),
        mesh=mesh,
        compiler_params=pltpu.CompilerParams(use_tc_tiling_on_sc=False),
        scratch_types=[
            pltpu.VMEM((j_chunks, _CH), jnp.int32),
            pltpu.VMEM((j_chunks, _CH), jnp.int32),
            pltpu.VMEM((_NBUF, _CH, d), jnp.float32),
            pltpu.VMEM((_CH, d), jnp.float32),
            pltpu.SemaphoreType.DMA,
            pltpu.SemaphoreType.DMA,
            pltpu.VMEM_SHARED((acc_rows, d), jnp.float32),
        ],
    )
    return fn(table, src3, dst3)


_BLK = 2048


def _tc_layer(p, ew, eb, cb):
    half = p.shape[0] // 2
    din = p.shape[1]
    dout = cb.shape[1]
    nb = half // _BLK

    def body(p0_ref, p1_ref, ew_ref, eb_ref, cb_ref, y_ref):
        w = _leaky(0.01 * ew_ref[...] + eb_ref[...])
        g = p0_ref[...] + p1_ref[...]
        y_ref[...] = _leaky(
            jnp.dot(g, w, precision=_HI, preferred_element_type=jnp.float32)
            + cb_ref[...])

    return pl.pallas_call(
        body,
        grid=(nb,),
        in_specs=[
            pl.BlockSpec((_BLK, din), lambda i: (i, 0)),
            pl.BlockSpec((_BLK, din), lambda i, _o=nb: (i + _o, 0)),
            pl.BlockSpec(ew.shape, lambda i: (0, 0)),
            pl.BlockSpec(eb.shape, lambda i: (0, 0)),
            pl.BlockSpec(cb.shape, lambda i: (0, 0)),
        ],
        out_specs=pl.BlockSpec((_BLK, dout), lambda i: (i, 0)),
        out_shape=jax.ShapeDtypeStruct((half, dout), jnp.float32),
    )(p, p, ew, eb, cb)


def _tc_finish(p, ew, eb, cb, w1, b1, w2, b2, batch, n):
    half = p.shape[0] // 2
    din = p.shape[1]
    dim1 = cb.shape[1]
    nc = b2.shape[1]
    nb = half // _BLK

    def body(p0_ref, p1_ref, ew_ref, eb_ref, cb_ref, w1_ref, b1_ref, w2_ref,
             b2_ref, o_ref, em_acc):
        i = pl.program_id(0)

        @pl.when(i == 0)
        def _init():
            em_acc[...] = jnp.zeros_like(em_acc)

        w = _leaky(0.01 * ew_ref[...] + eb_ref[...])
        g = p0_ref[...] + p1_ref[...]
        y = _leaky(
            jnp.dot(g, w, precision=_HI, preferred_element_type=jnp.float32)
            + cb_ref[...])
        ridx = i * _BLK + lax.broadcasted_iota(jnp.int32, (_BLK, 1), 0)
        for b in range(batch):
            m = (ridx >= b * n) & (ridx < (b + 1) * n)
            part = jnp.sum(jnp.where(m, y, 0.0), axis=0, keepdims=True)
            em_acc[b:b + 1, :] += part

        @pl.when(i == nb - 1)
        def _done():
            em = em_acc[...]
            h = _leaky(
                jnp.dot(em, w1_ref[...], precision=_HI,
                        preferred_element_type=jnp.float32) + b1_ref[...])
            o_ref[...] = jnp.dot(h, w2_ref[...], precision=_HI,
                                 preferred_element_type=jnp.float32) + b2_ref[...]

    return pl.pallas_call(
        body,
        grid=(nb,),
        in_specs=[
            pl.BlockSpec((_BLK, din), lambda i: (i, 0)),
            pl.BlockSpec((_BLK, din), lambda i, _o=nb: (i + _o, 0)),
            pl.BlockSpec(ew.shape, lambda i: (0, 0)),
            pl.BlockSpec(eb.shape, lambda i: (0, 0)),
            pl.BlockSpec(cb.shape, lambda i: (0, 0)),
            pl.BlockSpec(w1.shape, lambda i: (0, 0)),
            pl.BlockSpec(b1.shape, lambda i: (0, 0)),
            pl.BlockSpec(w2.shape, lambda i: (0, 0)),
            pl.BlockSpec(b2.shape, lambda i: (0, 0)),
        ],
        out_specs=pl.BlockSpec((batch, nc), lambda i: (0, 0)),
        out_shape=jax.ShapeDtypeStruct((batch, nc), jnp.float32),
        scratch_shapes=[pltpu.VMEM((batch, dim1), jnp.float32)],
    )(p, p, ew, eb, cb, w1, b1, w2, b2)


def kernel(x, edge_index, edge_w0, edge_b0, edge_w1, edge_b1,
           bias0, bias1, cls_w1, cls_b1, cls_w2, cls_b2):
    B, N, d = x.shape
    E = edge_index.shape[-1]
    dim1 = bias0.shape[0]
    rows_tot = B * N
    tot_edges = B * E

    ei = edge_index.astype(jnp.int32)
    offs = (jnp.arange(B, dtype=jnp.int32) * N)[:, None]
    src = (ei[:, 0, :] + offs).reshape(-1)
    dst = (ei[:, 1, :] + offs).reshape(-1)

    j_chunks = -(-tot_edges // (_NW * _CH))
    j_chunks = -(-j_chunks // _NBUF) * _NBUF
    pad = _NW * j_chunks * _CH - tot_edges
    src3 = jnp.concatenate(
        [src, jnp.zeros((pad,), jnp.int32)]).reshape(_NW, j_chunks, _CH)
    dst3 = jnp.concatenate(
        [dst, jnp.full((pad,), rows_tot, jnp.int32)]).reshape(_NW, j_chunks, _CH)

    ew0 = edge_w0.reshape(d, dim1)
    eb0 = edge_b0.reshape(d, dim1)
    ew1 = edge_w1.reshape(dim1, dim1)
    eb1 = edge_b1.reshape(dim1, dim1)

    table0 = x.reshape(rows_tot, d)
    p1 = _segment_accumulate(table0, src3, dst3, rows_tot, d, j_chunks)
    y1 = _tc_layer(p1, ew0, eb0, bias0.reshape(1, dim1))
    p2 = _segment_accumulate(y1, src3, dst3, rows_tot, dim1, j_chunks)
    out = _tc_finish(p2, ew1, eb1, bias1.reshape(1, dim1),
                     cls_w1, cls_b1.reshape(1, dim1),
                     cls_w2, cls_b2.reshape(1, -1), B, N)
    return out


# speedup vs baseline: 16.3935x; 1.2363x over previous
import jax
import jax.numpy as jnp
from jax import lax
from jax.experimental import pallas as pl
from jax.experimental.pallas import tpu as pltpu
from jax.experimental.pallas import tpu_sc as plsc

_NC = 2
_NS = 16
_NW = _NC * _NS
_CH = 128
_NBUF = 8

_HI = lax.Precision.HIGHEST


def _leaky(v):
    return jnp.where(v >= 0, v, 0.01 * v)


def _segment_accumulate(table, src3, dst3, rows_tot, d, j_chunks):
    acc_rows = (rows_tot // (_NS * _CH) + 1) * (_NS * _CH)
    zrows = acc_rows // _NS
    ngroups = j_chunks // _NBUF
    mesh = plsc.VectorSubcoreMesh(core_axis_name="c", subcore_axis_name="s",
                                  num_cores=_NC, num_subcores=_NS)

    def body(table_hbm, src_hbm, dst_hbm, out_hbm,
             sidx_v, didx_v, rows_v, zbuf_v, gsem, ssem, gsem2, ssem2, acc_sh):
        cid = lax.axis_index("c")
        sid = lax.axis_index("s")
        wid = sid * _NC + cid

        zv = jnp.zeros((16,), jnp.float32)
        for i in range(_CH):
            zbuf_v[i] = zv
        zd = [pltpu.async_copy(
                  zbuf_v, acc_sh.at[pl.ds(sid * zrows + t * _CH, _CH)], gsem)
              for t in range(zrows // _CH)]
        for dsc in zd:
            dsc.wait()

        pltpu.sync_copy(src_hbm.at[wid], sidx_v)
        pltpu.sync_copy(dst_hbm.at[wid], didx_v)
        plsc.subcore_barrier()

        def pair(g2, carry):
            base0 = g2 * 2 * _NBUF
            base1 = base0 + _NBUF
            gd0 = [pltpu.async_copy(table_hbm.at[sidx_v.at[base0 + b]],
                                    rows_v.at[0, b], gsem)
                   for b in range(_NBUF)]
            gd1 = [pltpu.async_copy(table_hbm.at[sidx_v.at[base1 + b]],
                                    rows_v.at[1, b], gsem2)
                   for b in range(_NBUF)]
            for dsc in gd0:
                dsc.wait()
            sd0 = [pltpu.async_copy(rows_v.at[0, b],
                                    acc_sh.at[didx_v.at[base0 + b]],
                                    ssem, add=True)
                   for b in range(_NBUF)]
            for dsc in gd1:
                dsc.wait()
            sd1 = [pltpu.async_copy(rows_v.at[1, b],
                                    acc_sh.at[didx_v.at[base1 + b]],
                                    ssem2, add=True)
                   for b in range(_NBUF)]
            for dsc in sd0:
                dsc.wait()
            for dsc in sd1:
                dsc.wait()
            return carry
        lax.fori_loop(0, ngroups // 2, pair, 0)
        plsc.subcore_barrier()

        pltpu.sync_copy(acc_sh.at[pl.ds(sid * zrows, zrows)],
                        out_hbm.at[pl.ds(cid * acc_rows + sid * zrows, zrows)])

    fn = pl.kernel(
        body,
        out_type=jax.ShapeDtypeStruct((_NC * acc_rows, d), jnp.float32),
        mesh=mesh,
        compiler_params=pltpu.CompilerParams(use_tc_tiling_on_sc=False),
        scratch_types=[
            pltpu.VMEM((j_chunks, _CH), jnp.int32),
            pltpu.VMEM((j_chunks, _CH), jnp.int32),
            pltpu.VMEM((2, _NBUF, _CH, d), jnp.float32),
            pltpu.VMEM((_CH, d), jnp.float32),
            pltpu.SemaphoreType.DMA,
            pltpu.SemaphoreType.DMA,
            pltpu.SemaphoreType.DMA,
            pltpu.SemaphoreType.DMA,
            pltpu.VMEM_SHARED((acc_rows, d), jnp.float32),
        ],
    )
    return fn(table, src3, dst3)


def _blockdiag(w, slots):
    zw = jnp.zeros_like(w)
    rows = [jnp.concatenate([w if j == i else zw for j in range(slots)], axis=1)
            for i in range(slots)]
    return jnp.concatenate(rows, axis=0)


def _tc_layer(p, ew, eb, cb):
    half = p.shape[0] // 2
    d = ew.shape[0]
    slots = 128 // d

    def body(p_ref, ew_ref, eb_ref, cb_ref, y_ref):
        w = _leaky(0.01 * ew_ref[...] + eb_ref[...])
        w8 = _blockdiag(w, slots)
        cbw = jnp.concatenate([cb_ref[...]] * slots, axis=1)
        pfull = p_ref[...]
        g = pfull[:half] + pfull[half:]
        y_ref[...] = _leaky(
            jnp.dot(g, w8, precision=_HI, preferred_element_type=jnp.float32)
            + cbw)

    return pl.pallas_call(
        body, out_shape=jax.ShapeDtypeStruct((half, 128), jnp.float32),
    )(p, ew, eb, cb)


def _tc_finish(p, ew, eb, cb, w1, b1, w2, b2, batch, n):
    half = p.shape[0] // 2
    d = ew.shape[0]
    slots = 128 // d
    npack = n // slots
    nc = b2.shape[1]

    def body(p_ref, ew_ref, eb_ref, cb_ref, w1_ref, b1_ref, w2_ref, b2_ref,
             o_ref):
        w = _leaky(0.01 * ew_ref[...] + eb_ref[...])
        w8 = _blockdiag(w, slots)
        cbw = jnp.concatenate([cb_ref[...]] * slots, axis=1)
        pfull = p_ref[...]
        g = pfull[:half] + pfull[half:]
        y = _leaky(
            jnp.dot(g, w8, precision=_HI, preferred_element_type=jnp.float32)
            + cbw)
        ems = []
        for b in range(batch):
            s = jnp.sum(y[b * npack:(b + 1) * npack], axis=0, keepdims=True)
            ems.append(sum(s[:, t * d:(t + 1) * d] for t in range(slots)))
        em = jnp.concatenate(ems, axis=0)
        h = _leaky(
            jnp.dot(em, w1_ref[...], precision=_HI,
                    preferred_element_type=jnp.float32) + b1_ref[...])
        o_ref[...] = jnp.dot(h, w2_ref[...], precision=_HI,
                             preferred_element_type=jnp.float32) + b2_ref[...]

    return pl.pallas_call(
        body, out_shape=jax.ShapeDtypeStruct((batch, nc), jnp.float32),
    )(p, ew, eb, cb, w1, b1, w2, b2)


def kernel(x, edge_index, edge_w0, edge_b0, edge_w1, edge_b1,
           bias0, bias1, cls_w1, cls_b1, cls_w2, cls_b2):
    B, N, d = x.shape
    E = edge_index.shape[-1]
    dim1 = bias0.shape[0]
    rows_tot = B * N
    tot_edges = B * E

    ei = edge_index.astype(jnp.int32)
    offs = (jnp.arange(B, dtype=jnp.int32) * N)[:, None]
    src = (ei[:, 0, :] + offs).reshape(-1)
    dst = (ei[:, 1, :] + offs).reshape(-1)

    j_chunks = -(-tot_edges // (_NW * _CH))
    j_chunks = -(-j_chunks // _NBUF) * _NBUF
    pad = _NW * j_chunks * _CH - tot_edges
    src3 = jnp.concatenate(
        [src, jnp.zeros((pad,), jnp.int32)]).reshape(_NW, j_chunks, _CH)
    dst3 = jnp.concatenate(
        [dst, jnp.full((pad,), rows_tot, jnp.int32)]).reshape(_NW, j_chunks, _CH)

    ew0 = edge_w0.reshape(d, dim1)
    eb0 = edge_b0.reshape(d, dim1)
    ew1 = edge_w1.reshape(dim1, dim1)
    eb1 = edge_b1.reshape(dim1, dim1)

    table0 = x.reshape(rows_tot, d)
    p1 = _segment_accumulate(table0, src3, dst3, rows_tot, d, j_chunks)
    pk1 = p1.reshape(p1.shape[0] * d // 128, 128)
    yk1 = _tc_layer(pk1, ew0, eb0, bias0.reshape(1, dim1))
    y1 = yk1.reshape(yk1.shape[0] * 128 // dim1, dim1)
    p2 = _segment_accumulate(y1, src3, dst3, rows_tot, dim1, j_chunks)
    pk2 = p2.reshape(p2.shape[0] * dim1 // 128, 128)
    out = _tc_finish(pk2, ew1, eb1, bias1.reshape(1, dim1),
                     cls_w1, cls_b1.reshape(1, dim1),
                     cls_w2, cls_b2.reshape(1, -1), B, N)
    return out


# speedup vs baseline: 19.7984x; 1.2077x over previous
import jax
import jax.numpy as jnp
from jax import lax
from jax.experimental import pallas as pl
from jax.experimental.pallas import tpu as pltpu
from jax.experimental.pallas import tpu_sc as plsc

_NC = 2
_NS = 16
_NW = _NC * _NS
_CH = 128
_NBUF = 8

_HI = lax.Precision.HIGHEST


def _leaky(v):
    return jnp.where(v >= 0, v, 0.01 * v)


def _acc_rows(rows_tot):
    return (rows_tot // (_NS * _CH) + 1) * (_NS * _CH)


def _segment_accumulate(table, srcc, dstc, rows_tot, d, j0, j1):
    acc_rows = _acc_rows(rows_tot)
    zrows = acc_rows // _NS
    jmax = max(j0, j1)
    assert j0 % _NBUF == 0 and j1 % _NBUF == 0
    assert (j0 // _NBUF) % 2 == (j1 // _NBUF) % 2 == 1
    mesh = plsc.VectorSubcoreMesh(core_axis_name="c", subcore_axis_name="s",
                                  num_cores=_NC, num_subcores=_NS)

    def body(table_hbm, src_hbm, dst_hbm, out_hbm,
             sidx_v, didx_v, rows_v, zbuf_v, gsem, ssem, gsem2, ssem2, acc_sh):
        cid = lax.axis_index("c")
        sid = lax.axis_index("s")

        zv = jnp.zeros((16,), jnp.float32)
        for i in range(_CH):
            zbuf_v[i] = zv
        zd = [pltpu.async_copy(
                  zbuf_v, acc_sh.at[pl.ds(sid * zrows + t * _CH, _CH)], gsem)
              for t in range(zrows // _CH)]
        for dsc in zd:
            dsc.wait()

        base_w = jnp.where(cid == 0, sid * j0, 16 * j0 + sid * j1)
        jmin = min(j0, j1)
        pltpu.sync_copy(src_hbm.at[pl.ds(base_w, jmin)],
                        sidx_v.at[pl.ds(0, jmin)])
        pltpu.sync_copy(dst_hbm.at[pl.ds(base_w, jmin)],
                        didx_v.at[pl.ds(0, jmin)])

        @pl.when(cid == (0 if j0 > j1 else 1))
        def _stage_rest():
            pltpu.sync_copy(src_hbm.at[pl.ds(base_w + jmin, jmax - jmin)],
                            sidx_v.at[pl.ds(jmin, jmax - jmin)])
            pltpu.sync_copy(dst_hbm.at[pl.ds(base_w + jmin, jmax - jmin)],
                            didx_v.at[pl.ds(jmin, jmax - jmin)])
        plsc.subcore_barrier()

        def do_group(base, bank, gsm, ssm):
            gd = [pltpu.async_copy(table_hbm.at[sidx_v.at[base + b]],
                                   rows_v.at[bank, b], gsm)
                  for b in range(_NBUF)]
            def scat():
                for dsc in gd:
                    dsc.wait()
                return [pltpu.async_copy(rows_v.at[bank, b],
                                         acc_sh.at[didx_v.at[base + b]],
                                         ssm, add=True)
                        for b in range(_NBUF)]
            return scat

        def pair(g2, carry):
            base0 = g2 * 2 * _NBUF
            s0 = do_group(base0, 0, gsem, ssem)
            s1 = do_group(base0 + _NBUF, 1, gsem2, ssem2)
            sd0 = s0()
            sd1 = s1()
            for dsc in sd0 + sd1:
                dsc.wait()
            return carry
        npairs = jnp.where(cid == 0, j0 // (2 * _NBUF), j1 // (2 * _NBUF))
        lax.fori_loop(0, npairs, pair, 0)
        tbase = npairs * 2 * _NBUF
        for dsc in do_group(tbase, 0, gsem, ssem)():
            dsc.wait()
        plsc.subcore_barrier()

        pltpu.sync_copy(acc_sh.at[pl.ds(sid * zrows, zrows)],
                        out_hbm.at[pl.ds(cid * acc_rows + sid * zrows, zrows)])

    fn = pl.kernel(
        body,
        out_type=jax.ShapeDtypeStruct((_NC * acc_rows, d), jnp.float32),
        mesh=mesh,
        compiler_params=pltpu.CompilerParams(use_tc_tiling_on_sc=False),
        scratch_types=[
            pltpu.VMEM((jmax, _CH), jnp.int32),
            pltpu.VMEM((jmax, _CH), jnp.int32),
            pltpu.VMEM((2, _NBUF, _CH, d), jnp.float32),
            pltpu.VMEM((_CH, d), jnp.float32),
            pltpu.SemaphoreType.DMA,
            pltpu.SemaphoreType.DMA,
            pltpu.SemaphoreType.DMA,
            pltpu.SemaphoreType.DMA,
            pltpu.VMEM_SHARED((acc_rows, d), jnp.float32),
        ],
    )
    return fn(table, srcc, dstc)


def _blockdiag(w, slots):
    zw = jnp.zeros_like(w)
    rows = [jnp.concatenate([w if j == i else zw for j in range(slots)], axis=1)
            for i in range(slots)]
    return jnp.concatenate(rows, axis=0)


def _tc_layer(p, ew, eb, cb):
    half = p.shape[0] // 2
    d = ew.shape[0]
    slots = 128 // d

    def body(p_ref, ew_ref, eb_ref, cb_ref, y_ref):
        w = _leaky(0.01 * ew_ref[...] + eb_ref[...])
        w8 = _blockdiag(w, slots)
        cbw = jnp.concatenate([cb_ref[...]] * slots, axis=1)
        pfull = p_ref[...]
        g = pfull[:half] + pfull[half:]
        y_ref[...] = _leaky(
            jnp.dot(g, w8, precision=_HI, preferred_element_type=jnp.float32)
            + cbw)

    return pl.pallas_call(
        body, out_shape=jax.ShapeDtypeStruct((half, 128), jnp.float32),
    )(p, ew, eb, cb)


def _tc_finish(p, ew, eb, cb, w1, b1, w2, b2, batch, n):
    half = p.shape[0] // 2
    d = ew.shape[0]
    slots = 128 // d
    npack = n // slots
    nc = b2.shape[1]

    def body(p_ref, ew_ref, eb_ref, cb_ref, w1_ref, b1_ref, w2_ref, b2_ref,
             o_ref):
        w = _leaky(0.01 * ew_ref[...] + eb_ref[...])
        w8 = _blockdiag(w, slots)
        cbw = jnp.concatenate([cb_ref[...]] * slots, axis=1)
        pfull = p_ref[...]
        g = pfull[:half] + pfull[half:]
        y = _leaky(
            jnp.dot(g, w8, precision=_HI, preferred_element_type=jnp.float32)
            + cbw)
        ems = []
        for b in range(batch):
            s = jnp.sum(y[b * npack:(b + 1) * npack], axis=0, keepdims=True)
            ems.append(sum(s[:, t * d:(t + 1) * d] for t in range(slots)))
        em = jnp.concatenate(ems, axis=0)
        h = _leaky(
            jnp.dot(em, w1_ref[...], precision=_HI,
                    preferred_element_type=jnp.float32) + b1_ref[...])
        o_ref[...] = jnp.dot(h, w2_ref[...], precision=_HI,
                             preferred_element_type=jnp.float32) + b2_ref[...]

    return pl.pallas_call(
        body, out_shape=jax.ShapeDtypeStruct((batch, nc), jnp.float32),
    )(p, ew, eb, cb, w1, b1, w2, b2)


def kernel(x, edge_index, edge_w0, edge_b0, edge_w1, edge_b1,
           bias0, bias1, cls_w1, cls_b1, cls_w2, cls_b2):
    B, N, d = x.shape
    E = edge_index.shape[-1]
    dim1 = bias0.shape[0]
    rows_tot = B * N
    tot_edges = B * E

    acc = _acc_rows(rows_tot)
    ei = edge_index.astype(jnp.int32)
    offs = (jnp.arange(B, dtype=jnp.int32) * N)[:, None, None]
    es = ei + offs
    gran = _NS * _CH * _NBUF
    epad = -(-E // gran) * gran - E
    srcc = jnp.pad(es[:, 0, :], ((0, 0), (0, epad)),
                   constant_values=rows_tot).reshape(-1, _CH)
    dstc = jnp.pad(es[:, 1, :], ((0, 0), (0, epad)),
                   constant_values=rows_tot).reshape(-1, _CH)
    tot_ch = srcc.shape[0]
    per_tec = tot_ch // _NS
    j0 = (int(per_tec * 0.75) // (2 * _NBUF)) * 2 * _NBUF + _NBUF
    j1 = per_tec - j0

    ew0 = edge_w0.reshape(d, dim1)
    eb0 = edge_b0.reshape(d, dim1)
    ew1 = edge_w1.reshape(dim1, dim1)
    eb1 = edge_b1.reshape(dim1, dim1)

    table0 = jnp.pad(x.reshape(rows_tot, d), ((0, acc - rows_tot), (0, 0)))
    p1 = _segment_accumulate(table0, srcc, dstc, rows_tot, d, j0, j1)
    pk1 = p1.reshape(p1.shape[0] * d // 128, 128)
    yk1 = _tc_layer(pk1, ew0, eb0, bias0.reshape(1, dim1))
    y1 = yk1.reshape(yk1.shape[0] * 128 // dim1, dim1)
    p2 = _segment_accumulate(y1, srcc, dstc, rows_tot, dim1, j0, j1)
    pk2 = p2.reshape(p2.shape[0] * dim1 // 128, 128)
    out = _tc_finish(pk2, ew1, eb1, bias1.reshape(1, dim1),
                     cls_w1, cls_b1.reshape(1, dim1),
                     cls_w2, cls_b2.reshape(1, -1), B, N)
    return out


# speedup vs baseline: 25.5743x; 1.2917x over previous
import jax
import jax.numpy as jnp
from jax import lax
from jax.experimental import pallas as pl
from jax.experimental.pallas import tpu as pltpu
from jax.experimental.pallas import tpu_sc as plsc

_NC = 2
_NS = 16
_NW = _NC * _NS
_CH = 128
_NBUF = 8
_CORE0_FRAC = 0.5

_HI = lax.Precision.HIGHEST


def _leaky(v):
    return jnp.where(v >= 0, v, 0.01 * v)


def _acc_rows(rows_tot):
    return (rows_tot // (_NS * _CH) + 1) * (_NS * _CH)


def _segment_accumulate(table, srcc, dstc, rows_tot, d, j0, j1):
    acc_rows = _acc_rows(rows_tot)
    zrows = acc_rows // _NS
    jmax = max(j0, j1)
    assert j0 % _NBUF == 0 and j1 % _NBUF == 0
    mesh = plsc.VectorSubcoreMesh(core_axis_name="c", subcore_axis_name="s",
                                  num_cores=_NC, num_subcores=_NS)

    def body(table_hbm, src_hbm, dst_hbm, out_hbm,
             sidx_v, didx_v, rows_v, zbuf_v, gsem, ssem, gsem2, ssem2,
             acc_sh, tab_sh):
        cid = lax.axis_index("c")
        sid = lax.axis_index("s")

        zv = jnp.zeros((16,), jnp.float32)
        for i in range(_CH):
            zbuf_v[i] = zv
        zd = [pltpu.async_copy(
                  zbuf_v, acc_sh.at[pl.ds(sid * zrows + t * _CH, _CH)], gsem)
              for t in range(zrows // _CH)]
        for dsc in zd:
            dsc.wait()

        @pl.when(cid == 1)
        def _stage_table():
            pltpu.sync_copy(table_hbm.at[pl.ds(sid * zrows, zrows)],
                            tab_sh.at[pl.ds(sid * zrows, zrows)])

        base_w = jnp.where(cid == 0, sid * j0, 16 * j0 + sid * j1)
        jmin = min(j0, j1)
        pltpu.sync_copy(src_hbm.at[pl.ds(base_w, jmin)],
                        sidx_v.at[pl.ds(0, jmin)])
        pltpu.sync_copy(dst_hbm.at[pl.ds(base_w, jmin)],
                        didx_v.at[pl.ds(0, jmin)])

        if jmax > jmin:
            @pl.when(cid == (0 if j0 > j1 else 1))
            def _stage_rest():
                pltpu.sync_copy(src_hbm.at[pl.ds(base_w + jmin, jmax - jmin)],
                                sidx_v.at[pl.ds(jmin, jmax - jmin)])
                pltpu.sync_copy(dst_hbm.at[pl.ds(base_w + jmin, jmax - jmin)],
                                didx_v.at[pl.ds(jmin, jmax - jmin)])
        plsc.subcore_barrier()

        def do_group(tab, base, bank, gsm, ssm):
            gd = [pltpu.async_copy(tab.at[sidx_v.at[base + b]],
                                   rows_v.at[bank, b], gsm)
                  for b in range(_NBUF)]
            def scat():
                for dsc in gd:
                    dsc.wait()
                return [pltpu.async_copy(rows_v.at[bank, b],
                                         acc_sh.at[didx_v.at[base + b]],
                                         ssm, add=True)
                        for b in range(_NBUF)]
            return scat

        def run_chunks(tab, j):
            def pair(g2, carry):
                base0 = g2 * 2 * _NBUF
                s0 = do_group(tab, base0, 0, gsem, ssem)
                s1 = do_group(tab, base0 + _NBUF, 1, gsem2, ssem2)
                sd0 = s0()
                sd1 = s1()
                for dsc in sd0 + sd1:
                    dsc.wait()
                return carry
            npairs = j // (2 * _NBUF)
            if npairs:
                lax.fori_loop(0, npairs, pair, 0)
            if j % (2 * _NBUF):
                for dsc in do_group(tab, npairs * 2 * _NBUF, 0, gsem, ssem)():
                    dsc.wait()

        @pl.when(cid == 0)
        def _run0():
            run_chunks(table_hbm, j0)

        @pl.when(cid == 1)
        def _run1():
            run_chunks(tab_sh, j1)
        plsc.subcore_barrier()

        pltpu.sync_copy(acc_sh.at[pl.ds(sid * zrows, zrows)],
                        out_hbm.at[pl.ds(cid * acc_rows + sid * zrows, zrows)])

    fn = pl.kernel(
        body,
        out_type=jax.ShapeDtypeStruct((_NC * acc_rows, d), jnp.float32),
        mesh=mesh,
        compiler_params=pltpu.CompilerParams(use_tc_tiling_on_sc=False),
        scratch_types=[
            pltpu.VMEM((jmax, _CH), jnp.int32),
            pltpu.VMEM((jmax, _CH), jnp.int32),
            pltpu.VMEM((2, _NBUF, _CH, d), jnp.float32),
            pltpu.VMEM((_CH, d), jnp.float32),
            pltpu.SemaphoreType.DMA,
            pltpu.SemaphoreType.DMA,
            pltpu.SemaphoreType.DMA,
            pltpu.SemaphoreType.DMA,
            pltpu.VMEM_SHARED((acc_rows, d), jnp.float32),
            pltpu.VMEM_SHARED((acc_rows, d), jnp.float32),
        ],
    )
    return fn(table, srcc, dstc)


def _blockdiag(w, slots):
    zw = jnp.zeros_like(w)
    rows = [jnp.concatenate([w if j == i else zw for j in range(slots)], axis=1)
            for i in range(slots)]
    return jnp.concatenate(rows, axis=0)


def _tc_layer(p, ew, eb, cb):
    half = p.shape[0] // 2
    d = ew.shape[0]
    slots = 128 // d

    def body(p_ref, ew_ref, eb_ref, cb_ref, y_ref):
        w = _leaky(0.01 * ew_ref[...] + eb_ref[...])
        w8 = _blockdiag(w, slots)
        cbw = jnp.concatenate([cb_ref[...]] * slots, axis=1)
        pfull = p_ref[...]
        g = pfull[:half] + pfull[half:]
        y_ref[...] = _leaky(
            jnp.dot(g, w8, precision=_HI, preferred_element_type=jnp.float32)
            + cbw)

    return pl.pallas_call(
        body, out_shape=jax.ShapeDtypeStruct((half, 128), jnp.float32),
    )(p, ew, eb, cb)


def _tc_finish(p, ew, eb, cb, w1, b1, w2, b2, batch, n):
    half = p.shape[0] // 2
    d = ew.shape[0]
    slots = 128 // d
    npack = n // slots
    nc = b2.shape[1]

    def body(p_ref, ew_ref, eb_ref, cb_ref, w1_ref, b1_ref, w2_ref, b2_ref,
             o_ref):
        w = _leaky(0.01 * ew_ref[...] + eb_ref[...])
        w8 = _blockdiag(w, slots)
        cbw = jnp.concatenate([cb_ref[...]] * slots, axis=1)
        pfull = p_ref[...]
        g = pfull[:half] + pfull[half:]
        y = _leaky(
            jnp.dot(g, w8, precision=_HI, preferred_element_type=jnp.float32)
            + cbw)
        ems = []
        for b in range(batch):
            s = jnp.sum(y[b * npack:(b + 1) * npack], axis=0, keepdims=True)
            ems.append(sum(s[:, t * d:(t + 1) * d] for t in range(slots)))
        em = jnp.concatenate(ems, axis=0)
        h = _leaky(
            jnp.dot(em, w1_ref[...], precision=_HI,
                    preferred_element_type=jnp.float32) + b1_ref[...])
        o_ref[...] = jnp.dot(h, w2_ref[...], precision=_HI,
                             preferred_element_type=jnp.float32) + b2_ref[...]

    return pl.pallas_call(
        body, out_shape=jax.ShapeDtypeStruct((batch, nc), jnp.float32),
    )(p, ew, eb, cb, w1, b1, w2, b2)


def kernel(x, edge_index, edge_w0, edge_b0, edge_w1, edge_b1,
           bias0, bias1, cls_w1, cls_b1, cls_w2, cls_b2):
    B, N, d = x.shape
    E = edge_index.shape[-1]
    dim1 = bias0.shape[0]
    rows_tot = B * N
    tot_edges = B * E

    acc = _acc_rows(rows_tot)
    ei = edge_index.astype(jnp.int32)
    offs = (jnp.arange(B, dtype=jnp.int32) * N)[:, None, None]
    es = ei + offs
    gran = _NS * _CH * _NBUF
    epad = -(-E // gran) * gran - E
    srcc = jnp.pad(es[:, 0, :], ((0, 0), (0, epad)),
                   constant_values=rows_tot).reshape(-1, _CH)
    dstc = jnp.pad(es[:, 1, :], ((0, 0), (0, epad)),
                   constant_values=rows_tot).reshape(-1, _CH)
    tot_ch = srcc.shape[0]
    per_tec = tot_ch // _NS
    j0 = max(_NBUF, (int(per_tec * _CORE0_FRAC) // _NBUF) * _NBUF)
    j1 = per_tec - j0

    ew0 = edge_w0.reshape(d, dim1)
    eb0 = edge_b0.reshape(d, dim1)
    ew1 = edge_w1.reshape(dim1, dim1)
    eb1 = edge_b1.reshape(dim1, dim1)

    table0 = jnp.pad(x.reshape(rows_tot, d), ((0, acc - rows_tot), (0, 0)))
    p1 = _segment_accumulate(table0, srcc, dstc, rows_tot, d, j0, j1)
    pk1 = p1.reshape(p1.shape[0] * d // 128, 128)
    yk1 = _tc_layer(pk1, ew0, eb0, bias0.reshape(1, dim1))
    y1 = yk1.reshape(yk1.shape[0] * 128 // dim1, dim1)
    p2 = _segment_accumulate(y1, srcc, dstc, rows_tot, dim1, j0, j1)
    pk2 = p2.reshape(p2.shape[0] * dim1 // 128, 128)
    out = _tc_finish(pk2, ew1, eb1, bias1.reshape(1, dim1),
                     cls_w1, cls_b1.reshape(1, dim1),
                     cls_w2, cls_b2.reshape(1, -1), B, N)
    return out


# speedup vs baseline: 28.7572x; 1.1245x over previous
import jax
import jax.numpy as jnp
from jax import lax
from jax.experimental import pallas as pl
from jax.experimental.pallas import tpu as pltpu
from jax.experimental.pallas import tpu_sc as plsc

_NC = 2
_NS = 16
_NW = _NC * _NS
_CH = 128
_NBUF = 8
_CORE0_FRAC = 0.5

_HI = lax.Precision.HIGHEST


def _leaky(v):
    return jnp.where(v >= 0, v, 0.01 * v)


def _acc_rows(rows_tot):
    return (rows_tot // (_NS * _CH) + 1) * (_NS * _CH)


def _segment_accumulate(table, srcc, dstc, rows_tot, d, j0, j1):
    acc_rows = _acc_rows(rows_tot)
    zrows = acc_rows // _NS
    jmax = max(j0, j1)
    assert j0 % (2 * _NBUF) == 0 and j1 % (2 * _NBUF) == 0
    mesh = plsc.VectorSubcoreMesh(core_axis_name="c", subcore_axis_name="s",
                                  num_cores=_NC, num_subcores=_NS)

    def body(table_hbm, src_hbm, dst_hbm, out_hbm,
             sidx_v, didx_v, rows_v, zbuf_v, gsem, ssem, gsem2, ssem2,
             acc_sh, tab_sh):
        cid = lax.axis_index("c")
        sid = lax.axis_index("s")

        zv = jnp.zeros((16,), jnp.float32)
        for i in range(_CH):
            zbuf_v[i] = zv
        zd = [pltpu.async_copy(
                  zbuf_v, acc_sh.at[pl.ds(sid * zrows + t * _CH, _CH)], gsem)
              for t in range(zrows // _CH)]
        for dsc in zd:
            dsc.wait()

        @pl.when(cid == 1)
        def _stage_table():
            pltpu.sync_copy(table_hbm.at[pl.ds(sid * zrows, zrows)],
                            tab_sh.at[pl.ds(sid * zrows, zrows)])

        half_ch = src_hbm.shape[0] // 2

        def stage(base, jh):
            sd = []
            for k in range(jh // _NBUF):
                for hbm, vm in ((src_hbm, sidx_v), (dst_hbm, didx_v)):
                    sd.append(pltpu.async_copy(
                        hbm.at[pl.ds(base + k * _NBUF, _NBUF)],
                        vm.at[pl.ds(2 * k * _NBUF, _NBUF)], gsem))
                    sd.append(pltpu.async_copy(
                        hbm.at[pl.ds(half_ch + base + k * _NBUF, _NBUF)],
                        vm.at[pl.ds((2 * k + 1) * _NBUF, _NBUF)], gsem))
            for dsc in sd:
                dsc.wait()

        @pl.when(cid == 0)
        def _stage0():
            stage(sid * (j0 // 2), j0 // 2)

        @pl.when(cid == 1)
        def _stage1():
            stage(16 * (j0 // 2) + sid * (j1 // 2), j1 // 2)
        plsc.subcore_barrier()

        def do_group(tab, base, bank, gsm, ssm):
            gd = [pltpu.async_copy(tab.at[sidx_v.at[base + b]],
                                   rows_v.at[bank, b], gsm)
                  for b in range(_NBUF)]
            def scat():
                for dsc in gd:
                    dsc.wait()
                return [pltpu.async_copy(rows_v.at[bank, b],
                                         acc_sh.at[didx_v.at[base + b]],
                                         ssm, add=True)
                        for b in range(_NBUF)]
            return scat

        def run_chunks(tab, j):
            def pair(g2, carry):
                base0 = g2 * 2 * _NBUF
                s0 = do_group(tab, base0, 0, gsem, ssem)
                s1 = do_group(tab, base0 + _NBUF, 1, gsem2, ssem2)
                sd0 = s0()
                sd1 = s1()
                for dsc in sd0 + sd1:
                    dsc.wait()
                return carry
            npairs = j // (2 * _NBUF)
            if npairs:
                lax.fori_loop(0, npairs, pair, 0)
            if j % (2 * _NBUF):
                for dsc in do_group(tab, npairs * 2 * _NBUF, 0, gsem, ssem)():
                    dsc.wait()

        @pl.when(cid == 0)
        def _run0():
            run_chunks(table_hbm, j0)

        @pl.when(cid == 1)
        def _run1():
            run_chunks(tab_sh, j1)
        plsc.subcore_barrier()

        pltpu.sync_copy(acc_sh.at[pl.ds(sid * zrows, zrows)],
                        out_hbm.at[pl.ds(cid * acc_rows + sid * zrows, zrows)])

    fn = pl.kernel(
        body,
        out_type=jax.ShapeDtypeStruct((_NC * acc_rows, d), jnp.float32),
        mesh=mesh,
        compiler_params=pltpu.CompilerParams(use_tc_tiling_on_sc=False),
        scratch_types=[
            pltpu.VMEM((jmax, _CH), jnp.int32),
            pltpu.VMEM((jmax, _CH), jnp.int32),
            pltpu.VMEM((2, _NBUF, _CH, d), jnp.float32),
            pltpu.VMEM((_CH, d), jnp.float32),
            pltpu.SemaphoreType.DMA,
            pltpu.SemaphoreType.DMA,
            pltpu.SemaphoreType.DMA,
            pltpu.SemaphoreType.DMA,
            pltpu.VMEM_SHARED((acc_rows, d), jnp.float32),
            pltpu.VMEM_SHARED((acc_rows, d), jnp.float32),
        ],
    )
    return fn(table, srcc, dstc)


def _blockdiag(w, slots):
    zw = jnp.zeros_like(w)
    rows = [jnp.concatenate([w if j == i else zw for j in range(slots)], axis=1)
            for i in range(slots)]
    return jnp.concatenate(rows, axis=0)


def _tc_layer(p, ew, eb, cb):
    half = p.shape[0] // 2
    d = ew.shape[0]
    slots = 128 // d

    def body(p_ref, ew_ref, eb_ref, cb_ref, y_ref):
        w = _leaky(0.01 * ew_ref[...] + eb_ref[...])
        w8 = _blockdiag(w, slots)
        cbw = jnp.concatenate([cb_ref[...]] * slots, axis=1)
        pfull = p_ref[...]
        g = pfull[:half] + pfull[half:]
        y_ref[...] = _leaky(
            jnp.dot(g, w8, precision=_HI, preferred_element_type=jnp.float32)
            + cbw)

    return pl.pallas_call(
        body, out_shape=jax.ShapeDtypeStruct((half, 128), jnp.float32),
    )(p, ew, eb, cb)


def _tc_finish(p, ew, eb, cb, w1, b1, w2, b2, batch, n):
    half = p.shape[0] // 2
    d = ew.shape[0]
    slots = 128 // d
    npack = n // slots
    nc = b2.shape[1]

    def body(p_ref, ew_ref, eb_ref, cb_ref, w1_ref, b1_ref, w2_ref, b2_ref,
             o_ref):
        w = _leaky(0.01 * ew_ref[...] + eb_ref[...])
        w8 = _blockdiag(w, slots)
        cbw = jnp.concatenate([cb_ref[...]] * slots, axis=1)
        pfull = p_ref[...]
        g = pfull[:half] + pfull[half:]
        y = _leaky(
            jnp.dot(g, w8, precision=_HI, preferred_element_type=jnp.float32)
            + cbw)
        ems = []
        for b in range(batch):
            s = jnp.sum(y[b * npack:(b + 1) * npack], axis=0, keepdims=True)
            ems.append(sum(s[:, t * d:(t + 1) * d] for t in range(slots)))
        em = jnp.concatenate(ems, axis=0)
        h = _leaky(
            jnp.dot(em, w1_ref[...], precision=_HI,
                    preferred_element_type=jnp.float32) + b1_ref[...])
        o_ref[...] = jnp.dot(h, w2_ref[...], precision=_HI,
                             preferred_element_type=jnp.float32) + b2_ref[...]

    return pl.pallas_call(
        body, out_shape=jax.ShapeDtypeStruct((batch, nc), jnp.float32),
    )(p, ew, eb, cb, w1, b1, w2, b2)


def kernel(x, edge_index, edge_w0, edge_b0, edge_w1, edge_b1,
           bias0, bias1, cls_w1, cls_b1, cls_w2, cls_b2):
    B, N, d = x.shape
    E = edge_index.shape[-1]
    dim1 = bias0.shape[0]
    rows_tot = B * N
    tot_edges = B * E

    acc = _acc_rows(rows_tot)
    ei = edge_index.astype(jnp.int32)
    offs = (jnp.arange(B, dtype=jnp.int32) * N)[:, None, None]
    es = ei + offs
    gran = _NS * _CH * _NBUF
    epad = -(-E // gran) * gran - E
    srcc = jnp.pad(es[:, 0, :], ((0, 0), (0, epad)),
                   constant_values=rows_tot).reshape(-1, _CH)
    dstc = jnp.pad(es[:, 1, :], ((0, 0), (0, epad)),
                   constant_values=rows_tot).reshape(-1, _CH)
    tot_ch = srcc.shape[0]
    per_tec = tot_ch // _NS
    j0 = max(2 * _NBUF, (int(per_tec * _CORE0_FRAC) // (2 * _NBUF)) * 2 * _NBUF)
    j1 = per_tec - j0

    ew0 = edge_w0.reshape(d, dim1)
    eb0 = edge_b0.reshape(d, dim1)
    ew1 = edge_w1.reshape(dim1, dim1)
    eb1 = edge_b1.reshape(dim1, dim1)

    table0 = jnp.pad(x.reshape(rows_tot, d), ((0, acc - rows_tot), (0, 0)))
    p1 = _segment_accumulate(table0, srcc, dstc, rows_tot, d, j0, j1)
    pk1 = p1.reshape(p1.shape[0] * d // 128, 128)
    yk1 = _tc_layer(pk1, ew0, eb0, bias0.reshape(1, dim1))
    y1 = yk1.reshape(yk1.shape[0] * 128 // dim1, dim1)
    p2 = _segment_accumulate(y1, srcc, dstc, rows_tot, dim1, j0, j1)
    pk2 = p2.reshape(p2.shape[0] * dim1 // 128, 128)
    out = _tc_finish(pk2, ew1, eb1, bias1.reshape(1, dim1),
                     cls_w1, cls_b1.reshape(1, dim1),
                     cls_w2, cls_b2.reshape(1, -1), B, N)
    return out


# speedup vs baseline: 30.7330x; 1.0687x over previous
import jax
import jax.numpy as jnp
from jax import lax
from jax.experimental import pallas as pl
from jax.experimental.pallas import tpu as pltpu
from jax.experimental.pallas import tpu_sc as plsc

_NC = 2
_NS = 16
_NW = _NC * _NS
_CH = 128
_NBUF = 8
_CORE0_FRAC = 0.6

_HI = lax.Precision.HIGHEST


def _leaky(v):
    return jnp.where(v >= 0, v, 0.01 * v)


def _acc_rows(rows_tot):
    return (rows_tot // (_NS * _CH) + 1) * (_NS * _CH)


def _segment_accumulate(table, srcc, dstc, rows_tot, d, j0, j1):
    acc_rows = _acc_rows(rows_tot)
    zrows = acc_rows // _NS
    jmax = max(j0, j1)
    assert j0 % (2 * _NBUF) == 0 and j1 % (2 * _NBUF) == 0
    mesh = plsc.VectorSubcoreMesh(core_axis_name="c", subcore_axis_name="s",
                                  num_cores=_NC, num_subcores=_NS)

    def body(table_hbm, src_hbm, dst_hbm, out_hbm,
             sidx_v, didx_v, rows_v, zbuf_v, gsem, ssem, gsem2, ssem2,
             acc_sh, tab_sh):
        cid = lax.axis_index("c")
        sid = lax.axis_index("s")

        zv = jnp.zeros((16,), jnp.float32)
        for i in range(_CH):
            zbuf_v[i] = zv
        zd = [pltpu.async_copy(
                  zbuf_v, acc_sh.at[pl.ds(sid * zrows + t * _CH, _CH)], gsem)
              for t in range(zrows // _CH)]
        for dsc in zd:
            dsc.wait()

        @pl.when(cid == 1)
        def _stage_table():
            pltpu.sync_copy(table_hbm.at[pl.ds(sid * zrows, zrows)],
                            tab_sh.at[pl.ds(sid * zrows, zrows)])

        half_ch = src_hbm.shape[0] // 2

        def stage(base, jh):
            sd = []
            for k in range(jh // _NBUF):
                for hbm, vm in ((src_hbm, sidx_v), (dst_hbm, didx_v)):
                    sd.append(pltpu.async_copy(
                        hbm.at[pl.ds(base + k * _NBUF, _NBUF)],
                        vm.at[pl.ds(2 * k * _NBUF, _NBUF)], gsem))
                    sd.append(pltpu.async_copy(
                        hbm.at[pl.ds(half_ch + base + k * _NBUF, _NBUF)],
                        vm.at[pl.ds((2 * k + 1) * _NBUF, _NBUF)], gsem))
            for dsc in sd:
                dsc.wait()

        @pl.when(cid == 0)
        def _stage0():
            stage(sid * (j0 // 2), j0 // 2)

        @pl.when(cid == 1)
        def _stage1():
            stage(16 * (j0 // 2) + sid * (j1 // 2), j1 // 2)
        plsc.subcore_barrier()

        def do_group(tab, base, bank, gsm, ssm):
            gd = [pltpu.async_copy(tab.at[sidx_v.at[base + b]],
                                   rows_v.at[bank, b], gsm)
                  for b in range(_NBUF)]
            def scat():
                for dsc in gd:
                    dsc.wait()
                return [pltpu.async_copy(rows_v.at[bank, b],
                                         acc_sh.at[didx_v.at[base + b]],
                                         ssm, add=True)
                        for b in range(_NBUF)]
            return scat

        def run_chunks(tab, j):
            def pair(g2, carry):
                base0 = g2 * 2 * _NBUF
                s0 = do_group(tab, base0, 0, gsem, ssem)
                s1 = do_group(tab, base0 + _NBUF, 1, gsem2, ssem2)
                sd0 = s0()
                sd1 = s1()
                for dsc in sd0 + sd1:
                    dsc.wait()
                return carry
            npairs = j // (2 * _NBUF)
            if npairs:
                lax.fori_loop(0, npairs, pair, 0)
            if j % (2 * _NBUF):
                for dsc in do_group(tab, npairs * 2 * _NBUF, 0, gsem, ssem)():
                    dsc.wait()

        @pl.when(cid == 0)
        def _run0():
            run_chunks(table_hbm, j0)

        @pl.when(cid == 1)
        def _run1():
            run_chunks(tab_sh, j1)
        plsc.subcore_barrier()

        pltpu.sync_copy(acc_sh.at[pl.ds(sid * zrows, zrows)],
                        out_hbm.at[pl.ds(cid * acc_rows + sid * zrows, zrows)])

    fn = pl.kernel(
        body,
        out_type=jax.ShapeDtypeStruct((_NC * acc_rows, d), jnp.float32),
        mesh=mesh,
        compiler_params=pltpu.CompilerParams(use_tc_tiling_on_sc=False),
        scratch_types=[
            pltpu.VMEM((jmax, _CH), jnp.int32),
            pltpu.VMEM((jmax, _CH), jnp.int32),
            pltpu.VMEM((2, _NBUF, _CH, d), jnp.float32),
            pltpu.VMEM((_CH, d), jnp.float32),
            pltpu.SemaphoreType.DMA,
            pltpu.SemaphoreType.DMA,
            pltpu.SemaphoreType.DMA,
            pltpu.SemaphoreType.DMA,
            pltpu.VMEM_SHARED((acc_rows, d), jnp.float32),
            pltpu.VMEM_SHARED((acc_rows, d), jnp.float32),
        ],
    )
    return fn(table, srcc, dstc)


def _tc_prep(ei4, x, rows_tot, ch_pb, acc_rows):
    B, _, ech, _ = ei4.shape
    N = x.shape[1]
    tot_ch = B * ch_pb

    def body(ei_ref, src_ref, dst_ref):
        padv = jnp.full((ch_pb - ech, _CH), rows_tot, jnp.int32)
        for b in range(B):
            src_ref[pl.ds(b * ch_pb, ech)] = ei_ref[b, 0] + b * N
            src_ref[pl.ds(b * ch_pb + ech, ch_pb - ech)] = padv
            dst_ref[pl.ds(b * ch_pb, ech)] = ei_ref[b, 1] + b * N
            dst_ref[pl.ds(b * ch_pb + ech, ch_pb - ech)] = padv

    return pl.pallas_call(
        body,
        out_shape=(
            jax.ShapeDtypeStruct((tot_ch, _CH), jnp.int32),
            jax.ShapeDtypeStruct((tot_ch, _CH), jnp.int32),
        ),
    )(ei4)


def _blockdiag(w, slots):
    zw = jnp.zeros_like(w)
    rows = [jnp.concatenate([w if j == i else zw for j in range(slots)], axis=1)
            for i in range(slots)]
    return jnp.concatenate(rows, axis=0)


def _tc_layer(p, ew, eb, cb):
    half = p.shape[0] // 2
    d = ew.shape[0]
    slots = 128 // d

    def body(p_ref, ew_ref, eb_ref, cb_ref, y_ref):
        w = _leaky(0.01 * ew_ref[...] + eb_ref[...])
        w8 = _blockdiag(w, slots)
        cbw = jnp.concatenate([cb_ref[...]] * slots, axis=1)
        pfull = p_ref[...]
        g = pfull[:half] + pfull[half:]
        y_ref[...] = _leaky(
            jnp.dot(g, w8, precision=_HI, preferred_element_type=jnp.float32)
            + cbw)

    return pl.pallas_call(
        body, out_shape=jax.ShapeDtypeStruct((half, 128), jnp.float32),
    )(p, ew, eb, cb)


def _tc_finish(p, ew, eb, cb, w1, b1, w2, b2, batch, n):
    half = p.shape[0] // 2
    d = ew.shape[0]
    slots = 128 // d
    npack = n // slots
    nc = b2.shape[1]

    def body(p_ref, ew_ref, eb_ref, cb_ref, w1_ref, b1_ref, w2_ref, b2_ref,
             o_ref):
        w = _leaky(0.01 * ew_ref[...] + eb_ref[...])
        w8 = _blockdiag(w, slots)
        cbw = jnp.concatenate([cb_ref[...]] * slots, axis=1)
        pfull = p_ref[...]
        g = pfull[:half] + pfull[half:]
        y = _leaky(
            jnp.dot(g, w8, precision=_HI, preferred_element_type=jnp.float32)
            + cbw)
        ems = []
        for b in range(batch):
            s = jnp.sum(y[b * npack:(b + 1) * npack], axis=0, keepdims=True)
            ems.append(sum(s[:, t * d:(t + 1) * d] for t in range(slots)))
        em = jnp.concatenate(ems, axis=0)
        h = _leaky(
            jnp.dot(em, w1_ref[...], precision=_HI,
                    preferred_element_type=jnp.float32) + b1_ref[...])
        o_ref[...] = jnp.dot(h, w2_ref[...], precision=_HI,
                             preferred_element_type=jnp.float32) + b2_ref[...]

    return pl.pallas_call(
        body, out_shape=jax.ShapeDtypeStruct((batch, nc), jnp.float32),
    )(p, ew, eb, cb, w1, b1, w2, b2)


def kernel(x, edge_index, edge_w0, edge_b0, edge_w1, edge_b1,
           bias0, bias1, cls_w1, cls_b1, cls_w2, cls_b2):
    B, N, d = x.shape
    E = edge_index.shape[-1]
    dim1 = bias0.shape[0]
    rows_tot = B * N
    tot_edges = B * E

    acc = _acc_rows(rows_tot)
    ei = edge_index.astype(jnp.int32)
    gran = _NS * _CH * _NBUF
    ch_pb = (-(-E // gran) * gran) // _CH
    ei4 = ei.reshape(B, 2, E // _CH, _CH)
    srcc, dstc = _tc_prep(ei4, x, rows_tot, ch_pb, acc)
    table0 = jnp.pad(x.reshape(rows_tot, d), ((0, acc - rows_tot), (0, 0)))
    tot_ch = srcc.shape[0]
    per_tec = tot_ch // _NS
    j0 = max(2 * _NBUF, (int(per_tec * _CORE0_FRAC) // (2 * _NBUF)) * 2 * _NBUF)
    j1 = per_tec - j0

    ew0 = edge_w0.reshape(d, dim1)
    eb0 = edge_b0.reshape(d, dim1)
    ew1 = edge_w1.reshape(dim1, dim1)
    eb1 = edge_b1.reshape(dim1, dim1)

    p1 = _segment_accumulate(table0, srcc, dstc, rows_tot, d, j0, j1)
    pk1 = p1.reshape(p1.shape[0] * d // 128, 128)
    yk1 = _tc_layer(pk1, ew0, eb0, bias0.reshape(1, dim1))
    y1 = yk1.reshape(yk1.shape[0] * 128 // dim1, dim1)
    p2 = _segment_accumulate(y1, srcc, dstc, rows_tot, dim1, j0, j1)
    pk2 = p2.reshape(p2.shape[0] * dim1 // 128, 128)
    out = _tc_finish(pk2, ew1, eb1, bias1.reshape(1, dim1),
                     cls_w1, cls_b1.reshape(1, dim1),
                     cls_w2, cls_b2.reshape(1, -1), B, N)
    return out


# speedup vs baseline: 34.5283x; 1.1235x over previous
import jax
import jax.numpy as jnp
from jax import lax
from jax.experimental import pallas as pl
from jax.experimental.pallas import tpu as pltpu
from jax.experimental.pallas import tpu_sc as plsc

_NC = 2
_NS = 16
_NW = _NC * _NS
_CH = 128
_NBUF = 8
_CORE0_FRAC = 0.6

_HI = lax.Precision.HIGHEST


def _leaky(v):
    return jnp.where(v >= 0, v, 0.01 * v)


def _acc_rows(rows_tot):
    return (rows_tot // (_NS * _CH) + 1) * (_NS * _CH)


def _segment_accumulate(table, srcc, dstc, rows_tot, d, j0, j1):
    acc_rows = _acc_rows(rows_tot)
    zrows = acc_rows // _NS
    jmax = max(j0, j1)
    assert j0 % (2 * _NBUF) == 0 and j1 % (2 * _NBUF) == 0
    mesh = plsc.VectorSubcoreMesh(core_axis_name="c", subcore_axis_name="s",
                                  num_cores=_NC, num_subcores=_NS)

    def body(table_hbm, src_hbm, dst_hbm, out_hbm,
             sidx_v, didx_v, rows_v, zbuf_v, gsem, ssem, gsem2, ssem2,
             acc_sh, tab_sh):
        cid = lax.axis_index("c")
        sid = lax.axis_index("s")

        zv = jnp.zeros((16,), jnp.float32)
        for i in range(_CH):
            zbuf_v[i] = zv
        zd = [pltpu.async_copy(
                  zbuf_v, acc_sh.at[pl.ds(sid * zrows + t * _CH, _CH)], gsem)
              for t in range(zrows // _CH)]
        for dsc in zd:
            dsc.wait()

        trt = table.shape[0] // _NS

        @pl.when(cid == 1)
        def _stage_table():
            pltpu.sync_copy(table_hbm.at[pl.ds(sid * trt, trt)],
                            tab_sh.at[pl.ds(sid * trt, trt)])

        half_ch = src_hbm.shape[0] // 2

        def stage(base, jh):
            sd = []
            for k in range(jh // _NBUF):
                for hbm, vm in ((src_hbm, sidx_v), (dst_hbm, didx_v)):
                    sd.append(pltpu.async_copy(
                        hbm.at[pl.ds(base + k * _NBUF, _NBUF)],
                        vm.at[pl.ds(2 * k * _NBUF, _NBUF)], gsem))
                    sd.append(pltpu.async_copy(
                        hbm.at[pl.ds(half_ch + base + k * _NBUF, _NBUF)],
                        vm.at[pl.ds((2 * k + 1) * _NBUF, _NBUF)], gsem))
            for dsc in sd:
                dsc.wait()

        @pl.when(cid == 0)
        def _stage0():
            stage(sid * (j0 // 2), j0 // 2)

        @pl.when(cid == 1)
        def _stage1():
            stage(16 * (j0 // 2) + sid * (j1 // 2), j1 // 2)
        plsc.subcore_barrier()

        def do_group(tab, base, bank, gsm, ssm):
            gd = [pltpu.async_copy(tab.at[sidx_v.at[base + b]],
                                   rows_v.at[bank, b], gsm)
                  for b in range(_NBUF)]
            def scat():
                for dsc in gd:
                    dsc.wait()
                return [pltpu.async_copy(rows_v.at[bank, b],
                                         acc_sh.at[didx_v.at[base + b]],
                                         ssm, add=True)
                        for b in range(_NBUF)]
            return scat

        def run_chunks(tab, j):
            def pair(g2, carry):
                base0 = g2 * 2 * _NBUF
                s0 = do_group(tab, base0, 0, gsem, ssem)
                s1 = do_group(tab, base0 + _NBUF, 1, gsem2, ssem2)
                sd0 = s0()
                sd1 = s1()
                for dsc in sd0 + sd1:
                    dsc.wait()
                return carry
            npairs = j // (2 * _NBUF)
            if npairs:
                lax.fori_loop(0, npairs, pair, 0)
            if j % (2 * _NBUF):
                for dsc in do_group(tab, npairs * 2 * _NBUF, 0, gsem, ssem)():
                    dsc.wait()

        @pl.when(cid == 0)
        def _run0():
            run_chunks(table_hbm, j0)

        @pl.when(cid == 1)
        def _run1():
            run_chunks(tab_sh, j1)
        plsc.subcore_barrier()

        pltpu.sync_copy(acc_sh.at[pl.ds(sid * zrows, zrows)],
                        out_hbm.at[pl.ds(cid * acc_rows + sid * zrows, zrows)])

    fn = pl.kernel(
        body,
        out_type=jax.ShapeDtypeStruct((_NC * acc_rows, d), jnp.float32),
        mesh=mesh,
        compiler_params=pltpu.CompilerParams(use_tc_tiling_on_sc=False),
        scratch_types=[
            pltpu.VMEM((jmax, _CH), jnp.int32),
            pltpu.VMEM((jmax, _CH), jnp.int32),
            pltpu.VMEM((2, _NBUF, _CH, d), jnp.float32),
            pltpu.VMEM((_CH, d), jnp.float32),
            pltpu.SemaphoreType.DMA,
            pltpu.SemaphoreType.DMA,
            pltpu.SemaphoreType.DMA,
            pltpu.SemaphoreType.DMA,
            pltpu.VMEM_SHARED((acc_rows, d), jnp.float32),
            pltpu.VMEM_SHARED((acc_rows, d), jnp.float32),
        ],
    )
    return fn(table, srcc, dstc)


def _tc_prep(ei, x, rows_tot, ch_pb, acc_rows):
    B, _, E = ei.shape
    N, d = x.shape[1], x.shape[2]
    ech = E // _CH
    slots = 128 // d
    npk = N // slots
    tot_ch = B * ch_pb
    trows = acc_rows * d // 128

    def body(ei_ref, src_ref, dst_ref):
        padv_s = jnp.zeros((ch_pb - ech, _CH), jnp.int32)
        padv_d = jnp.full((ch_pb - ech, _CH), rows_tot, jnp.int32)
        for b in range(B):
            src_ref[pl.ds(b * ch_pb, ech)] = (
                ei_ref[b, 0].reshape(ech, _CH) + b * N)
            src_ref[pl.ds(b * ch_pb + ech, ch_pb - ech)] = padv_s
            dst_ref[pl.ds(b * ch_pb, ech)] = (
                ei_ref[b, 1].reshape(ech, _CH) + b * N)
            dst_ref[pl.ds(b * ch_pb + ech, ch_pb - ech)] = padv_d

    return pl.pallas_call(
        body,
        out_shape=(
            jax.ShapeDtypeStruct((tot_ch, _CH), jnp.int32),
            jax.ShapeDtypeStruct((tot_ch, _CH), jnp.int32),
        ),
    )(ei)


def _blockdiag(w, slots):
    zw = jnp.zeros_like(w)
    rows = [jnp.concatenate([w if j == i else zw for j in range(slots)], axis=1)
            for i in range(slots)]
    return jnp.concatenate(rows, axis=0)


def _tc_layer(p, ew, eb, cb):
    half = p.shape[0] // 2
    d = ew.shape[0]
    slots = 128 // d

    def body(p_ref, ew_ref, eb_ref, cb_ref, y_ref):
        w = _leaky(0.01 * ew_ref[...] + eb_ref[...])
        w8 = _blockdiag(w, slots)
        cbw = jnp.concatenate([cb_ref[...]] * slots, axis=1)
        pfull = p_ref[...]
        g = pfull[:half] + pfull[half:]
        y_ref[...] = _leaky(
            jnp.dot(g, w8, precision=_HI, preferred_element_type=jnp.float32)
            + cbw)

    return pl.pallas_call(
        body, out_shape=jax.ShapeDtypeStruct((half, 128), jnp.float32),
    )(p, ew, eb, cb)


def _tc_finish(p, ew, eb, cb, w1, b1, w2, b2, batch, n):
    half = p.shape[0] // 2
    d = ew.shape[0]
    slots = 128 // d
    npack = n // slots
    nc = b2.shape[1]

    def body(p_ref, ew_ref, eb_ref, cb_ref, w1_ref, b1_ref, w2_ref, b2_ref,
             o_ref):
        w = _leaky(0.01 * ew_ref[...] + eb_ref[...])
        w8 = _blockdiag(w, slots)
        cbw = jnp.concatenate([cb_ref[...]] * slots, axis=1)
        pfull = p_ref[...]
        g = pfull[:half] + pfull[half:]
        y = _leaky(
            jnp.dot(g, w8, precision=_HI, preferred_element_type=jnp.float32)
            + cbw)
        ems = []
        for b in range(batch):
            s = jnp.sum(y[b * npack:(b + 1) * npack], axis=0, keepdims=True)
            ems.append(sum(s[:, t * d:(t + 1) * d] for t in range(slots)))
        em = jnp.concatenate(ems, axis=0)
        h = _leaky(
            jnp.dot(em, w1_ref[...], precision=_HI,
                    preferred_element_type=jnp.float32) + b1_ref[...])
        o_ref[...] = jnp.dot(h, w2_ref[...], precision=_HI,
                             preferred_element_type=jnp.float32) + b2_ref[...]

    return pl.pallas_call(
        body, out_shape=jax.ShapeDtypeStruct((batch, nc), jnp.float32),
    )(p, ew, eb, cb, w1, b1, w2, b2)


def kernel(x, edge_index, edge_w0, edge_b0, edge_w1, edge_b1,
           bias0, bias1, cls_w1, cls_b1, cls_w2, cls_b2):
    B, N, d = x.shape
    E = edge_index.shape[-1]
    dim1 = bias0.shape[0]
    rows_tot = B * N
    tot_edges = B * E

    acc = _acc_rows(rows_tot)
    ei = edge_index.astype(jnp.int32)
    gran = _NS * _CH * _NBUF
    ch_pb = (-(-E // gran) * gran) // _CH
    srcc, dstc = _tc_prep(ei, x, rows_tot, ch_pb, acc)
    table0 = x.reshape(rows_tot, d)
    tot_ch = srcc.shape[0]
    per_tec = tot_ch // _NS
    j0 = max(2 * _NBUF, (int(per_tec * _CORE0_FRAC) // (2 * _NBUF)) * 2 * _NBUF)
    j1 = per_tec - j0

    ew0 = edge_w0.reshape(d, dim1)
    eb0 = edge_b0.reshape(d, dim1)
    ew1 = edge_w1.reshape(dim1, dim1)
    eb1 = edge_b1.reshape(dim1, dim1)

    p1 = _segment_accumulate(table0, srcc, dstc, rows_tot, d, j0, j1)
    pk1 = p1.reshape(p1.shape[0] * d // 128, 128)
    yk1 = _tc_layer(pk1, ew0, eb0, bias0.reshape(1, dim1))
    y1 = yk1.reshape(yk1.shape[0] * 128 // dim1, dim1)
    p2 = _segment_accumulate(y1, srcc, dstc, rows_tot, dim1, j0, j1)
    pk2 = p2.reshape(p2.shape[0] * dim1 // 128, 128)
    out = _tc_finish(pk2, ew1, eb1, bias1.reshape(1, dim1),
                     cls_w1, cls_b1.reshape(1, dim1),
                     cls_w2, cls_b2.reshape(1, -1), B, N)
    return out
